# Initial kernel scaffold; baseline (speedup 1.0000x reference)
import jax
import jax.numpy as jnp
from jax.experimental import pallas as pl


def kernel(l, m, s, conv_l_pre_down_w, conv_l_pre_down_scale, conv_l_pre_down_bias, conv_l_post_down_w, conv_l_post_down_scale, conv_l_post_down_bias, conv_m_w, conv_m_scale, conv_m_bias, conv_s_pre_up_w, conv_s_pre_up_scale, conv_s_pre_up_bias, conv_s_post_up_w, conv_s_post_up_scale, conv_s_post_up_bias, trans0_w, trans0_scale, trans0_bias, trans1_w, trans1_scale, trans1_bias, trans2_w, trans2_scale, trans2_bias, trans3_w, trans3_scale, trans3_bias):
    raise NotImplementedError("write your pallas kernel here")


# speedup vs baseline: 1.3808x; 1.3808x over previous
import functools

import jax
import jax.numpy as jnp
from jax.experimental import pallas as pl
from jax.experimental.pallas import tpu as pltpu


_PARALLEL = pltpu.CompilerParams(dimension_semantics=("parallel",))


def _ru(x, m):
    return (x + m - 1) // m * m


def _resident(shape):
    nd = len(shape)
    return pl.BlockSpec(tuple(shape), lambda t, _nd=nd: (0,) * _nd)


def _conv_body(x_ref, w_ref, sb_ref, o_ref, xs_ref, *, offsets, tm, tmh_r, relu):
    base = pl.program_id(0) * tm
    xs_ref[...] = x_ref[:, pl.ds(base, tmh_r)]
    acc = jnp.zeros(o_ref.shape, jnp.float32)
    for k, off in enumerate(offsets):
        acc += jnp.dot(w_ref[k], xs_ref[:, off:off + tm],
                       preferred_element_type=jnp.float32)
    y = acc * sb_ref[0:1, :].T + sb_ref[1:2, :].T
    if relu:
        y = jnp.maximum(y, 0.0)
    o_ref[...] = y.astype(o_ref.dtype)


def _conv_bn(x, w, scale, bias, relu=True, tm=512):
    c, n, h, wd = x.shape
    kh, kw, cin, cout = w.shape
    pad = kh // 2
    hp, wp = h + 2 * pad, wd + 2 * pad
    mp = n * hp * wp
    l_out = _ru(mp, tm)
    nt = l_out // tm
    mr = pad * wp + pad
    tmh_r = _ru(tm + 2 * mr, 128)
    lh = mr + l_out + (tmh_r - tm - mr)

    xp = jnp.pad(x, ((0, 0), (0, 0), (pad, pad), (pad, pad))).reshape(c, mp)
    slab = jnp.pad(xp, ((0, 0), (mr, lh - mp - mr)))
    offsets = [ki * wp + kj for ki in range(kh) for kj in range(kw)]
    wf = jnp.transpose(w.reshape(kh * kw, cin, cout), (0, 2, 1)).astype(jnp.bfloat16)
    sb = jnp.stack([scale, bias]).astype(jnp.float32)

    out = pl.pallas_call(
        functools.partial(_conv_body, offsets=offsets, tm=tm, tmh_r=tmh_r,
                          relu=relu),
        grid=(nt,),
        in_specs=[_resident(slab.shape), _resident(wf.shape), _resident(sb.shape)],
        out_specs=pl.BlockSpec((cout, tm), lambda t: (0, t)),
        out_shape=jax.ShapeDtypeStruct((cout, l_out), jnp.bfloat16),
        scratch_shapes=[pltpu.VMEM((cin, tmh_r), jnp.bfloat16)],
        compiler_params=_PARALLEL,
    )(slab, wf, sb)
    out = out[:, :mp].reshape(cout, n, hp, wp)
    return out[:, :, pad:pad + h, pad:pad + wd]


def _pool_conv_body(x_ref, w_ref, sb_ref, o_ref, ps_ref, *, offsets, tm, tmh_r):
    base = pl.program_id(0) * tm
    xw = x_ref[:, :, pl.ds(base, tmh_r)].astype(jnp.float32)
    pooled = jnp.max(xw, axis=0) + jnp.sum(xw, axis=0) * 0.25
    ps_ref[...] = pooled.astype(jnp.bfloat16)
    acc = jnp.zeros(o_ref.shape, jnp.float32)
    for k, off in enumerate(offsets):
        acc += jnp.dot(w_ref[k], ps_ref[:, off:off + tm],
                       preferred_element_type=jnp.float32)
    y = jnp.maximum(acc * sb_ref[0:1, :].T + sb_ref[1:2, :].T, 0.0)
    o_ref[...] = y.astype(o_ref.dtype)


def _pool_then_conv(x, w, scale, bias, tm=512):
    c, n, h2, w2 = x.shape
    h, wd = h2 // 2, w2 // 2
    kh, kw, cin, cout = w.shape
    pad = kh // 2
    hp, wp = h + 2 * pad, wd + 2 * pad
    mp = n * hp * wp
    l_out = _ru(mp, tm)
    nt = l_out // tm
    mr = pad * wp + pad
    tmh_r = _ru(tm + 2 * mr, 128)
    lh = mr + l_out + (tmh_r - tm - mr)

    phases = jnp.stack([x[:, :, di::2, dj::2]
                        for di in range(2) for dj in range(2)], axis=0)
    phases = jnp.pad(phases, ((0, 0), (0, 0), (0, 0), (pad, pad), (pad, pad)))
    phases = phases.reshape(4, c, mp)
    slab = jnp.pad(phases, ((0, 0), (0, 0), (mr, lh - mp - mr)))

    offsets = [ki * wp + kj for ki in range(kh) for kj in range(kw)]
    wf = jnp.transpose(w.reshape(kh * kw, cin, cout), (0, 2, 1)).astype(jnp.bfloat16)
    sb = jnp.stack([scale, bias]).astype(jnp.float32)

    out = pl.pallas_call(
        functools.partial(_pool_conv_body, offsets=offsets, tm=tm, tmh_r=tmh_r),
        grid=(nt,),
        in_specs=[_resident(slab.shape), _resident(wf.shape), _resident(sb.shape)],
        out_specs=pl.BlockSpec((cout, tm), lambda t: (0, t)),
        out_shape=jax.ShapeDtypeStruct((cout, l_out), jnp.bfloat16),
        scratch_shapes=[pltpu.VMEM((cin, tmh_r), jnp.bfloat16)],
        compiler_params=_PARALLEL,
    )(slab, wf, sb)
    out = out[:, :mp].reshape(cout, n, hp, wp)
    return out[:, :, pad:pad + h, pad:pad + wd]


def _trans0_body(l_ref, m_ref, s_ref, w_ref, sb_ref, o_ref):
    acc = jnp.dot(w_ref[0], l_ref[...], preferred_element_type=jnp.float32)
    acc += jnp.dot(w_ref[1], m_ref[...], preferred_element_type=jnp.float32)
    acc += jnp.dot(w_ref[2], s_ref[...], preferred_element_type=jnp.float32)
    y = jnp.maximum(acc * sb_ref[0:1, :].T + sb_ref[1:2, :].T, 0.0)
    o_ref[...] = y.astype(o_ref.dtype)


def _fuse_body(a_ref, w3_ref, b3_ref, l_ref, m_ref, s_ref, o_ref):
    attn = jnp.dot(w3_ref[...], a_ref[...],
                   preferred_element_type=jnp.float32) + b3_ref[...]
    amax = jnp.max(attn, axis=0, keepdims=True)
    e = jnp.exp(attn - amax)
    wgt = e / jnp.sum(e, axis=0, keepdims=True)
    o_ref[...] = (wgt[0:1, :] * l_ref[...].astype(jnp.float32)
                  + wgt[1:2, :] * m_ref[...].astype(jnp.float32)
                  + wgt[2:3, :] * s_ref[...].astype(jnp.float32))


def _mtile(c, tm):
    return pl.BlockSpec((c, tm), lambda t: (0, t))


def kernel(l, m, s, conv_l_pre_down_w, conv_l_pre_down_scale, conv_l_pre_down_bias, conv_l_post_down_w, conv_l_post_down_scale, conv_l_post_down_bias, conv_m_w, conv_m_scale, conv_m_bias, conv_s_pre_up_w, conv_s_pre_up_scale, conv_s_pre_up_bias, conv_s_post_up_w, conv_s_post_up_scale, conv_s_post_up_bias, trans0_w, trans0_scale, trans0_bias, trans1_w, trans1_scale, trans1_bias, trans2_w, trans2_scale, trans2_bias, trans3_w, trans3_scale, trans3_bias):
    n, c, hm, wm = m.shape
    lc = jnp.transpose(l, (1, 0, 2, 3)).astype(jnp.bfloat16)
    mc = jnp.transpose(m, (1, 0, 2, 3)).astype(jnp.bfloat16)
    sc = jnp.transpose(s, (1, 0, 2, 3)).astype(jnp.bfloat16)

    l1 = _conv_bn(lc, conv_l_pre_down_w, conv_l_pre_down_scale, conv_l_pre_down_bias)
    l3 = _pool_then_conv(l1, conv_l_post_down_w, conv_l_post_down_scale,
                         conv_l_post_down_bias)
    m1 = _conv_bn(mc, conv_m_w, conv_m_scale, conv_m_bias)
    s1 = _conv_bn(sc, conv_s_pre_up_w, conv_s_pre_up_scale, conv_s_pre_up_bias)
    s2 = jnp.repeat(jnp.repeat(s1, 2, axis=2), 2, axis=3)
    s3 = _conv_bn(s2, conv_s_post_up_w, conv_s_post_up_scale, conv_s_post_up_bias)

    mm = n * hm * wm
    tm = 512
    nt = mm // tm
    lf = l3.reshape(c, mm)
    mf = m1.reshape(c, mm)
    sf = s3.reshape(c, mm)

    w0 = jnp.transpose(trans0_w.reshape(3, c, c), (0, 2, 1)).astype(jnp.bfloat16)
    sb0 = jnp.stack([trans0_scale, trans0_bias]).astype(jnp.float32)
    a0 = pl.pallas_call(
        _trans0_body,
        grid=(nt,),
        in_specs=[_mtile(c, tm), _mtile(c, tm), _mtile(c, tm),
                  _resident(w0.shape), _resident(sb0.shape)],
        out_specs=_mtile(c, tm),
        out_shape=jax.ShapeDtypeStruct((c, mm), jnp.bfloat16),
        compiler_params=_PARALLEL,
    )(lf, mf, sf, w0, sb0)

    a1 = _conv_bn(a0.reshape(c, n, hm, wm), trans1_w, trans1_scale, trans1_bias)
    a2 = _conv_bn(a1, trans2_w, trans2_scale, trans2_bias)

    w3 = (trans3_w.reshape(c, 3).T * trans3_scale.reshape(3, 1)).astype(jnp.bfloat16)
    b3 = trans3_bias.reshape(3, 1).astype(jnp.float32)
    out = pl.pallas_call(
        _fuse_body,
        grid=(nt,),
        in_specs=[_mtile(c, tm), _resident(w3.shape), _resident(b3.shape),
                  _mtile(c, tm), _mtile(c, tm), _mtile(c, tm)],
        out_specs=_mtile(c, tm),
        out_shape=jax.ShapeDtypeStruct((c, mm), jnp.float32),
        compiler_params=_PARALLEL,
    )(a2.reshape(c, mm), w3, b3, lf, mf, sf)

    return jnp.transpose(out.reshape(c, n, hm, wm), (1, 0, 2, 3))


# speedup vs baseline: 1.9944x; 1.4443x over previous
import functools

import jax
import jax.numpy as jnp
from jax.experimental import pallas as pl
from jax.experimental.pallas import tpu as pltpu


_PARALLEL = pltpu.CompilerParams(dimension_semantics=("parallel",))


def _ru(x, m):
    return (x + m - 1) // m * m


def _resident(shape):
    nd = len(shape)
    return pl.BlockSpec(tuple(shape), lambda t, _nd=nd: (0,) * _nd)


def _conv_body(x_ref, w_ref, sb_ref, o_ref, xs_ref, *, offsets, tm, tmh_r, relu):
    base = pl.program_id(0) * tm
    xs_ref[...] = x_ref[:, pl.ds(base, tmh_r)]
    acc = jnp.zeros(o_ref.shape, jnp.float32)
    for k, off in enumerate(offsets):
        acc += jnp.dot(w_ref[k], xs_ref[:, off:off + tm],
                       preferred_element_type=jnp.float32)
    y = acc * sb_ref[0:1, :].T + sb_ref[1:2, :].T
    if relu:
        y = jnp.maximum(y, 0.0)
    o_ref[...] = y.astype(o_ref.dtype)


def _conv_bn(x, w, scale, bias, relu=True, tm=512):
    c, n, h, wd = x.shape
    kh, kw, cin, cout = w.shape
    pad = kh // 2
    hp, wp = h + 2 * pad, wd + 2 * pad
    mp = n * hp * wp
    l_out = _ru(mp, tm)
    nt = l_out // tm
    mr = pad * wp + pad
    tmh_r = _ru(tm + 2 * mr, 128)
    lh = mr + l_out + (tmh_r - tm - mr)

    xp = jnp.pad(x, ((0, 0), (0, 0), (pad, pad), (pad, pad))).reshape(c, mp)
    slab = jnp.pad(xp, ((0, 0), (mr, lh - mp - mr)))
    offsets = [ki * wp + kj for ki in range(kh) for kj in range(kw)]
    wf = jnp.transpose(w.reshape(kh * kw, cin, cout), (0, 2, 1)).astype(jnp.bfloat16)
    sb = jnp.stack([scale, bias]).astype(jnp.float32)

    out = pl.pallas_call(
        functools.partial(_conv_body, offsets=offsets, tm=tm, tmh_r=tmh_r,
                          relu=relu),
        grid=(nt,),
        in_specs=[_resident(slab.shape), _resident(wf.shape), _resident(sb.shape)],
        out_specs=pl.BlockSpec((cout, tm), lambda t: (0, t)),
        out_shape=jax.ShapeDtypeStruct((cout, l_out), jnp.bfloat16),
        scratch_shapes=[pltpu.VMEM((cin, tmh_r), jnp.bfloat16)],
        compiler_params=_PARALLEL,
    )(slab, wf, sb)
    out = out[:, :mp].reshape(cout, n, hp, wp)
    return out[:, :, pad:pad + h, pad:pad + wd]


def _pool_body(x_ref, se_ref, so_ref, o_ref, *, c, rows):
    w2 = x_ref.shape[2]
    xw = x_ref[...].astype(jnp.float32).reshape(c, rows, 2, w2)
    hmax = jnp.max(xw, axis=2).reshape(c * rows, w2)
    hsum = jnp.sum(xw, axis=2).reshape(c * rows, w2)
    even = jnp.dot(hmax, se_ref[...], preferred_element_type=jnp.float32)
    odd = jnp.dot(hmax, so_ref[...], preferred_element_type=jnp.float32)
    psum = jnp.dot(hsum, se_ref[...] + so_ref[...],
                   preferred_element_type=jnp.float32)
    y = jnp.maximum(even, odd) + psum * 0.25
    o_ref[...] = y.reshape(c, rows, w2 // 2).astype(o_ref.dtype)


def _pool_then_conv(x, w, scale, bias, rows=8):
    c, n, h2, w2 = x.shape
    h, wd = h2 // 2, w2 // 2
    nt = (n * h) // rows

    eye = jnp.eye(wd, dtype=jnp.float32)
    zl = jnp.zeros((wd, wd), jnp.float32)
    se = jnp.stack([eye, zl], axis=1).reshape(w2, wd)
    so = jnp.stack([zl, eye], axis=1).reshape(w2, wd)

    pooled = pl.pallas_call(
        functools.partial(_pool_body, c=c, rows=rows),
        grid=(nt,),
        in_specs=[pl.BlockSpec((c, 2 * rows, w2), lambda t: (0, t, 0)),
                  _resident(se.shape), _resident(so.shape)],
        out_specs=pl.BlockSpec((c, rows, wd), lambda t: (0, t, 0)),
        out_shape=jax.ShapeDtypeStruct((c, n * h, wd), jnp.bfloat16),
        compiler_params=_PARALLEL,
    )(x.reshape(c, n * h2, w2), se, so)
    return _conv_bn(pooled.reshape(c, n, h, wd), w, scale, bias)


def _trans0_body(l_ref, m_ref, s_ref, w_ref, sb_ref, o_ref):
    acc = jnp.dot(w_ref[0], l_ref[...], preferred_element_type=jnp.float32)
    acc += jnp.dot(w_ref[1], m_ref[...], preferred_element_type=jnp.float32)
    acc += jnp.dot(w_ref[2], s_ref[...], preferred_element_type=jnp.float32)
    y = jnp.maximum(acc * sb_ref[0:1, :].T + sb_ref[1:2, :].T, 0.0)
    o_ref[...] = y.astype(o_ref.dtype)


def _fuse_body(a_ref, w3_ref, b3_ref, l_ref, m_ref, s_ref, o_ref):
    attn = jnp.dot(w3_ref[...], a_ref[...],
                   preferred_element_type=jnp.float32) + b3_ref[...]
    amax = jnp.max(attn, axis=0, keepdims=True)
    e = jnp.exp(attn - amax)
    wgt = e / jnp.sum(e, axis=0, keepdims=True)
    o_ref[...] = (wgt[0:1, :] * l_ref[...].astype(jnp.float32)
                  + wgt[1:2, :] * m_ref[...].astype(jnp.float32)
                  + wgt[2:3, :] * s_ref[...].astype(jnp.float32))


def _mtile(c, tm):
    return pl.BlockSpec((c, tm), lambda t: (0, t))


def kernel(l, m, s, conv_l_pre_down_w, conv_l_pre_down_scale, conv_l_pre_down_bias, conv_l_post_down_w, conv_l_post_down_scale, conv_l_post_down_bias, conv_m_w, conv_m_scale, conv_m_bias, conv_s_pre_up_w, conv_s_pre_up_scale, conv_s_pre_up_bias, conv_s_post_up_w, conv_s_post_up_scale, conv_s_post_up_bias, trans0_w, trans0_scale, trans0_bias, trans1_w, trans1_scale, trans1_bias, trans2_w, trans2_scale, trans2_bias, trans3_w, trans3_scale, trans3_bias):
    n, c, hm, wm = m.shape
    lc = jnp.transpose(l, (1, 0, 2, 3)).astype(jnp.bfloat16)
    mc = jnp.transpose(m, (1, 0, 2, 3)).astype(jnp.bfloat16)
    sc = jnp.transpose(s, (1, 0, 2, 3)).astype(jnp.bfloat16)

    l1 = _conv_bn(lc, conv_l_pre_down_w, conv_l_pre_down_scale, conv_l_pre_down_bias)
    l3 = _pool_then_conv(l1, conv_l_post_down_w, conv_l_post_down_scale,
                         conv_l_post_down_bias)
    m1 = _conv_bn(mc, conv_m_w, conv_m_scale, conv_m_bias)
    s1 = _conv_bn(sc, conv_s_pre_up_w, conv_s_pre_up_scale, conv_s_pre_up_bias)
    s2 = jnp.repeat(jnp.repeat(s1, 2, axis=2), 2, axis=3)
    s3 = _conv_bn(s2, conv_s_post_up_w, conv_s_post_up_scale, conv_s_post_up_bias)

    mm = n * hm * wm
    tm = 512
    nt = mm // tm
    lf = l3.reshape(c, mm)
    mf = m1.reshape(c, mm)
    sf = s3.reshape(c, mm)

    w0 = jnp.transpose(trans0_w.reshape(3, c, c), (0, 2, 1)).astype(jnp.bfloat16)
    sb0 = jnp.stack([trans0_scale, trans0_bias]).astype(jnp.float32)
    a0 = pl.pallas_call(
        _trans0_body,
        grid=(nt,),
        in_specs=[_mtile(c, tm), _mtile(c, tm), _mtile(c, tm),
                  _resident(w0.shape), _resident(sb0.shape)],
        out_specs=_mtile(c, tm),
        out_shape=jax.ShapeDtypeStruct((c, mm), jnp.bfloat16),
        compiler_params=_PARALLEL,
    )(lf, mf, sf, w0, sb0)

    a1 = _conv_bn(a0.reshape(c, n, hm, wm), trans1_w, trans1_scale, trans1_bias)
    a2 = _conv_bn(a1, trans2_w, trans2_scale, trans2_bias)

    w3 = (trans3_w.reshape(c, 3).T * trans3_scale.reshape(3, 1)).astype(jnp.bfloat16)
    b3 = trans3_bias.reshape(3, 1).astype(jnp.float32)
    out = pl.pallas_call(
        _fuse_body,
        grid=(nt,),
        in_specs=[_mtile(c, tm), _resident(w3.shape), _resident(b3.shape),
                  _mtile(c, tm), _mtile(c, tm), _mtile(c, tm)],
        out_specs=_mtile(c, tm),
        out_shape=jax.ShapeDtypeStruct((c, mm), jnp.float32),
        compiler_params=_PARALLEL,
    )(a2.reshape(c, mm), w3, b3, lf, mf, sf)

    return jnp.transpose(out.reshape(c, n, hm, wm), (1, 0, 2, 3))


# speedup vs baseline: 2.1714x; 1.0888x over previous
import functools

import jax
import jax.numpy as jnp
from jax.experimental import pallas as pl
from jax.experimental.pallas import tpu as pltpu


_PARALLEL = pltpu.CompilerParams(dimension_semantics=("parallel",))


def _ru(x, m):
    return (x + m - 1) // m * m


def _resident(shape):
    nd = len(shape)
    return pl.BlockSpec(tuple(shape), lambda t, _nd=nd: (0,) * _nd)


def _conv_body(x_ref, w_ref, sb_ref, o_ref, xs_ref, *, offsets, tm, tmh_r, relu):
    base = pl.program_id(0) * tm
    xs_ref[...] = x_ref[:, pl.ds(base, tmh_r)]
    acc = jnp.zeros(o_ref.shape, jnp.float32)
    for k, off in enumerate(offsets):
        acc += jnp.dot(w_ref[k], xs_ref[:, off:off + tm],
                       preferred_element_type=jnp.float32)
    y = acc * sb_ref[0:1, :].T + sb_ref[1:2, :].T
    if relu:
        y = jnp.maximum(y, 0.0)
    o_ref[...] = y.astype(o_ref.dtype)


def _conv_bn(x, w, scale, bias, relu=True, tm=1024):
    c, n, h, wd = x.shape
    kh, kw, cin, cout = w.shape
    pad = kh // 2
    hp, wp = h + 2 * pad, wd + 2 * pad
    mp = n * hp * wp
    l_out = _ru(mp, tm)
    nt = l_out // tm
    mr = pad * wp + pad
    tmh_r = _ru(tm + 2 * mr, 128)
    lh = mr + l_out + (tmh_r - tm - mr)

    xp = jnp.pad(x, ((0, 0), (0, 0), (pad, pad), (pad, pad))).reshape(c, mp)
    slab = jnp.pad(xp, ((0, 0), (mr, lh - mp - mr)))
    offsets = [ki * wp + kj for ki in range(kh) for kj in range(kw)]
    wf = jnp.transpose(w.reshape(kh * kw, cin, cout), (0, 2, 1)).astype(jnp.bfloat16)
    sb = jnp.stack([scale, bias]).astype(jnp.float32)

    out = pl.pallas_call(
        functools.partial(_conv_body, offsets=offsets, tm=tm, tmh_r=tmh_r,
                          relu=relu),
        grid=(nt,),
        in_specs=[_resident(slab.shape), _resident(wf.shape), _resident(sb.shape)],
        out_specs=pl.BlockSpec((cout, tm), lambda t: (0, t)),
        out_shape=jax.ShapeDtypeStruct((cout, l_out), jnp.bfloat16),
        scratch_shapes=[pltpu.VMEM((cin, tmh_r), jnp.bfloat16)],
        compiler_params=_PARALLEL,
    )(slab, wf, sb)
    out = out[:, :mp].reshape(cout, n, hp, wp)
    return out[:, :, pad:pad + h, pad:pad + wd]


def _pool_body(x_ref, se_ref, so_ref, o_ref, *, c, rows):
    w2 = x_ref.shape[2]
    xw = x_ref[...].astype(jnp.float32).reshape(c, rows, 2, w2)
    hmax = jnp.max(xw, axis=2).reshape(c * rows, w2)
    hsum = jnp.sum(xw, axis=2).reshape(c * rows, w2)
    even = jnp.dot(hmax, se_ref[...], preferred_element_type=jnp.float32)
    odd = jnp.dot(hmax, so_ref[...], preferred_element_type=jnp.float32)
    psum = jnp.dot(hsum, se_ref[...] + so_ref[...],
                   preferred_element_type=jnp.float32)
    y = jnp.maximum(even, odd) + psum * 0.25
    o_ref[...] = y.reshape(c, rows, w2 // 2).astype(o_ref.dtype)


def _pool_then_conv(x, w, scale, bias, rows=8):
    c, n, h2, w2 = x.shape
    h, wd = h2 // 2, w2 // 2
    nt = (n * h) // rows

    eye = jnp.eye(wd, dtype=jnp.float32)
    zl = jnp.zeros((wd, wd), jnp.float32)
    se = jnp.stack([eye, zl], axis=1).reshape(w2, wd)
    so = jnp.stack([zl, eye], axis=1).reshape(w2, wd)

    pooled = pl.pallas_call(
        functools.partial(_pool_body, c=c, rows=rows),
        grid=(nt,),
        in_specs=[pl.BlockSpec((c, 2 * rows, w2), lambda t: (0, t, 0)),
                  _resident(se.shape), _resident(so.shape)],
        out_specs=pl.BlockSpec((c, rows, wd), lambda t: (0, t, 0)),
        out_shape=jax.ShapeDtypeStruct((c, n * h, wd), jnp.bfloat16),
        compiler_params=_PARALLEL,
    )(x.reshape(c, n * h2, w2), se, so)
    return _conv_bn(pooled.reshape(c, n, h, wd), w, scale, bias)


def _trans0_body(l_ref, m_ref, s_ref, w_ref, sb_ref, o_ref):
    acc = jnp.dot(w_ref[0], l_ref[...], preferred_element_type=jnp.float32)
    acc += jnp.dot(w_ref[1], m_ref[...], preferred_element_type=jnp.float32)
    acc += jnp.dot(w_ref[2], s_ref[...], preferred_element_type=jnp.float32)
    y = jnp.maximum(acc * sb_ref[0:1, :].T + sb_ref[1:2, :].T, 0.0)
    o_ref[...] = y.astype(o_ref.dtype)


def _fuse_body(a_ref, w3_ref, b3_ref, l_ref, m_ref, s_ref, o_ref):
    attn = jnp.dot(w3_ref[...], a_ref[...],
                   preferred_element_type=jnp.float32) + b3_ref[...]
    amax = jnp.max(attn, axis=0, keepdims=True)
    e = jnp.exp(attn - amax)
    wgt = e / jnp.sum(e, axis=0, keepdims=True)
    o_ref[...] = (wgt[0:1, :] * l_ref[...].astype(jnp.float32)
                  + wgt[1:2, :] * m_ref[...].astype(jnp.float32)
                  + wgt[2:3, :] * s_ref[...].astype(jnp.float32))


def _mtile(c, tm):
    return pl.BlockSpec((c, tm), lambda t: (0, t))


def kernel(l, m, s, conv_l_pre_down_w, conv_l_pre_down_scale, conv_l_pre_down_bias, conv_l_post_down_w, conv_l_post_down_scale, conv_l_post_down_bias, conv_m_w, conv_m_scale, conv_m_bias, conv_s_pre_up_w, conv_s_pre_up_scale, conv_s_pre_up_bias, conv_s_post_up_w, conv_s_post_up_scale, conv_s_post_up_bias, trans0_w, trans0_scale, trans0_bias, trans1_w, trans1_scale, trans1_bias, trans2_w, trans2_scale, trans2_bias, trans3_w, trans3_scale, trans3_bias):
    n, c, hm, wm = m.shape
    lc = jnp.transpose(l, (1, 0, 2, 3)).astype(jnp.bfloat16)
    mc = jnp.transpose(m, (1, 0, 2, 3)).astype(jnp.bfloat16)
    sc = jnp.transpose(s, (1, 0, 2, 3)).astype(jnp.bfloat16)

    l1 = _conv_bn(lc, conv_l_pre_down_w, conv_l_pre_down_scale, conv_l_pre_down_bias)
    l3 = _pool_then_conv(l1, conv_l_post_down_w, conv_l_post_down_scale,
                         conv_l_post_down_bias)
    m1 = _conv_bn(mc, conv_m_w, conv_m_scale, conv_m_bias)
    s1 = _conv_bn(sc, conv_s_pre_up_w, conv_s_pre_up_scale, conv_s_pre_up_bias)
    s2 = jnp.repeat(jnp.repeat(s1, 2, axis=2), 2, axis=3)
    s3 = _conv_bn(s2, conv_s_post_up_w, conv_s_post_up_scale, conv_s_post_up_bias)

    mm = n * hm * wm
    tm = 512
    nt = mm // tm
    lf = l3.reshape(c, mm)
    mf = m1.reshape(c, mm)
    sf = s3.reshape(c, mm)

    w0 = jnp.transpose(trans0_w.reshape(3, c, c), (0, 2, 1)).astype(jnp.bfloat16)
    sb0 = jnp.stack([trans0_scale, trans0_bias]).astype(jnp.float32)
    a0 = pl.pallas_call(
        _trans0_body,
        grid=(nt,),
        in_specs=[_mtile(c, tm), _mtile(c, tm), _mtile(c, tm),
                  _resident(w0.shape), _resident(sb0.shape)],
        out_specs=_mtile(c, tm),
        out_shape=jax.ShapeDtypeStruct((c, mm), jnp.bfloat16),
        compiler_params=_PARALLEL,
    )(lf, mf, sf, w0, sb0)

    a1 = _conv_bn(a0.reshape(c, n, hm, wm), trans1_w, trans1_scale, trans1_bias)
    a2 = _conv_bn(a1, trans2_w, trans2_scale, trans2_bias)

    w3 = (trans3_w.reshape(c, 3).T * trans3_scale.reshape(3, 1)).astype(jnp.bfloat16)
    b3 = trans3_bias.reshape(3, 1).astype(jnp.float32)
    out = pl.pallas_call(
        _fuse_body,
        grid=(nt,),
        in_specs=[_mtile(c, tm), _resident(w3.shape), _resident(b3.shape),
                  _mtile(c, tm), _mtile(c, tm), _mtile(c, tm)],
        out_specs=_mtile(c, tm),
        out_shape=jax.ShapeDtypeStruct((c, mm), jnp.float32),
        compiler_params=_PARALLEL,
    )(a2.reshape(c, mm), w3, b3, lf, mf, sf)

    return jnp.transpose(out.reshape(c, n, hm, wm), (1, 0, 2, 3))


# speedup vs baseline: 2.2664x; 1.0437x over previous
import functools

import jax
import jax.numpy as jnp
from jax.experimental import pallas as pl
from jax.experimental.pallas import tpu as pltpu


_PARALLEL = pltpu.CompilerParams(dimension_semantics=("parallel",))


def _ru(x, m):
    return (x + m - 1) // m * m


def _resident(shape):
    nd = len(shape)
    return pl.BlockSpec(tuple(shape), lambda t, _nd=nd: (0,) * _nd)


def _conv_body(x_ref, w_ref, sb_ref, o_ref, xs_ref, *, offsets, tm, tmh_r, relu):
    base = pl.program_id(0) * tm
    xs_ref[...] = x_ref[:, pl.ds(base, tmh_r)]
    acc = jnp.zeros(o_ref.shape, jnp.float32)
    for k, off in enumerate(offsets):
        acc += jnp.dot(w_ref[k], xs_ref[:, off:off + tm],
                       preferred_element_type=jnp.float32)
    y = acc * sb_ref[0:1, :].T + sb_ref[1:2, :].T
    if relu:
        y = jnp.maximum(y, 0.0)
    o_ref[...] = y.astype(o_ref.dtype)


def _conv_bn(x, w, scale, bias, relu=True, tm=2048):
    c, n, h, wd = x.shape
    kh, kw, cin, cout = w.shape
    pad = kh // 2
    hp, wp = h + 2 * pad, wd + 2 * pad
    mp = n * hp * wp
    l_out = _ru(mp, tm)
    nt = l_out // tm
    mr = pad * wp + pad
    tmh_r = _ru(tm + 2 * mr, 128)
    lh = mr + l_out + (tmh_r - tm - mr)

    xp = jnp.pad(x, ((0, 0), (0, 0), (pad, pad), (pad, pad))).reshape(c, mp)
    slab = jnp.pad(xp, ((0, 0), (mr, lh - mp - mr)))
    offsets = [ki * wp + kj for ki in range(kh) for kj in range(kw)]
    wf = jnp.transpose(w.reshape(kh * kw, cin, cout), (0, 2, 1)).astype(jnp.bfloat16)
    sb = jnp.stack([scale, bias]).astype(jnp.float32)

    out = pl.pallas_call(
        functools.partial(_conv_body, offsets=offsets, tm=tm, tmh_r=tmh_r,
                          relu=relu),
        grid=(nt,),
        in_specs=[_resident(slab.shape), _resident(wf.shape), _resident(sb.shape)],
        out_specs=pl.BlockSpec((cout, tm), lambda t: (0, t)),
        out_shape=jax.ShapeDtypeStruct((cout, l_out), jnp.bfloat16),
        scratch_shapes=[pltpu.VMEM((cin, tmh_r), jnp.bfloat16)],
        compiler_params=_PARALLEL,
    )(slab, wf, sb)
    out = out[:, :mp].reshape(cout, n, hp, wp)
    return out[:, :, pad:pad + h, pad:pad + wd]


def _pool_body(x_ref, se_ref, so_ref, o_ref, *, c, rows):
    w2 = x_ref.shape[2]
    xw = x_ref[...].astype(jnp.float32).reshape(c, rows, 2, w2)
    hmax = jnp.max(xw, axis=2).reshape(c * rows, w2)
    hsum = jnp.sum(xw, axis=2).reshape(c * rows, w2)
    even = jnp.dot(hmax, se_ref[...], preferred_element_type=jnp.float32)
    odd = jnp.dot(hmax, so_ref[...], preferred_element_type=jnp.float32)
    psum = jnp.dot(hsum, se_ref[...] + so_ref[...],
                   preferred_element_type=jnp.float32)
    y = jnp.maximum(even, odd) + psum * 0.25
    o_ref[...] = y.reshape(c, rows, w2 // 2).astype(o_ref.dtype)


def _pool_then_conv(x, w, scale, bias, rows=8):
    c, n, h2, w2 = x.shape
    h, wd = h2 // 2, w2 // 2
    nt = (n * h) // rows

    eye = jnp.eye(wd, dtype=jnp.float32)
    zl = jnp.zeros((wd, wd), jnp.float32)
    se = jnp.stack([eye, zl], axis=1).reshape(w2, wd)
    so = jnp.stack([zl, eye], axis=1).reshape(w2, wd)

    pooled = pl.pallas_call(
        functools.partial(_pool_body, c=c, rows=rows),
        grid=(nt,),
        in_specs=[pl.BlockSpec((c, 2 * rows, w2), lambda t: (0, t, 0)),
                  _resident(se.shape), _resident(so.shape)],
        out_specs=pl.BlockSpec((c, rows, wd), lambda t: (0, t, 0)),
        out_shape=jax.ShapeDtypeStruct((c, n * h, wd), jnp.bfloat16),
        compiler_params=_PARALLEL,
    )(x.reshape(c, n * h2, w2), se, so)
    return _conv_bn(pooled.reshape(c, n, h, wd), w, scale, bias)


def _trans0_body(l_ref, m_ref, s_ref, w_ref, sb_ref, o_ref):
    acc = jnp.dot(w_ref[0], l_ref[...], preferred_element_type=jnp.float32)
    acc += jnp.dot(w_ref[1], m_ref[...], preferred_element_type=jnp.float32)
    acc += jnp.dot(w_ref[2], s_ref[...], preferred_element_type=jnp.float32)
    y = jnp.maximum(acc * sb_ref[0:1, :].T + sb_ref[1:2, :].T, 0.0)
    o_ref[...] = y.astype(o_ref.dtype)


def _fuse_body(a_ref, w3_ref, b3_ref, l_ref, m_ref, s_ref, o_ref):
    attn = jnp.dot(w3_ref[...], a_ref[...],
                   preferred_element_type=jnp.float32) + b3_ref[...]
    amax = jnp.max(attn, axis=0, keepdims=True)
    e = jnp.exp(attn - amax)
    wgt = e / jnp.sum(e, axis=0, keepdims=True)
    o_ref[...] = (wgt[0:1, :] * l_ref[...].astype(jnp.float32)
                  + wgt[1:2, :] * m_ref[...].astype(jnp.float32)
                  + wgt[2:3, :] * s_ref[...].astype(jnp.float32))


def _mtile(c, tm):
    return pl.BlockSpec((c, tm), lambda t: (0, t))


def kernel(l, m, s, conv_l_pre_down_w, conv_l_pre_down_scale, conv_l_pre_down_bias, conv_l_post_down_w, conv_l_post_down_scale, conv_l_post_down_bias, conv_m_w, conv_m_scale, conv_m_bias, conv_s_pre_up_w, conv_s_pre_up_scale, conv_s_pre_up_bias, conv_s_post_up_w, conv_s_post_up_scale, conv_s_post_up_bias, trans0_w, trans0_scale, trans0_bias, trans1_w, trans1_scale, trans1_bias, trans2_w, trans2_scale, trans2_bias, trans3_w, trans3_scale, trans3_bias):
    n, c, hm, wm = m.shape
    lc = jnp.transpose(l, (1, 0, 2, 3)).astype(jnp.bfloat16)
    mc = jnp.transpose(m, (1, 0, 2, 3)).astype(jnp.bfloat16)
    sc = jnp.transpose(s, (1, 0, 2, 3)).astype(jnp.bfloat16)

    l1 = _conv_bn(lc, conv_l_pre_down_w, conv_l_pre_down_scale, conv_l_pre_down_bias)
    l3 = _pool_then_conv(l1, conv_l_post_down_w, conv_l_post_down_scale,
                         conv_l_post_down_bias)
    m1 = _conv_bn(mc, conv_m_w, conv_m_scale, conv_m_bias)
    s1 = _conv_bn(sc, conv_s_pre_up_w, conv_s_pre_up_scale, conv_s_pre_up_bias)
    s2 = jnp.repeat(jnp.repeat(s1, 2, axis=2), 2, axis=3)
    s3 = _conv_bn(s2, conv_s_post_up_w, conv_s_post_up_scale, conv_s_post_up_bias)

    mm = n * hm * wm
    tm = 512
    nt = mm // tm
    lf = l3.reshape(c, mm)
    mf = m1.reshape(c, mm)
    sf = s3.reshape(c, mm)

    w0 = jnp.transpose(trans0_w.reshape(3, c, c), (0, 2, 1)).astype(jnp.bfloat16)
    sb0 = jnp.stack([trans0_scale, trans0_bias]).astype(jnp.float32)
    a0 = pl.pallas_call(
        _trans0_body,
        grid=(nt,),
        in_specs=[_mtile(c, tm), _mtile(c, tm), _mtile(c, tm),
                  _resident(w0.shape), _resident(sb0.shape)],
        out_specs=_mtile(c, tm),
        out_shape=jax.ShapeDtypeStruct((c, mm), jnp.bfloat16),
        compiler_params=_PARALLEL,
    )(lf, mf, sf, w0, sb0)

    a1 = _conv_bn(a0.reshape(c, n, hm, wm), trans1_w, trans1_scale, trans1_bias)
    a2 = _conv_bn(a1, trans2_w, trans2_scale, trans2_bias)

    w3 = (trans3_w.reshape(c, 3).T * trans3_scale.reshape(3, 1)).astype(jnp.bfloat16)
    b3 = trans3_bias.reshape(3, 1).astype(jnp.float32)
    out = pl.pallas_call(
        _fuse_body,
        grid=(nt,),
        in_specs=[_mtile(c, tm), _resident(w3.shape), _resident(b3.shape),
                  _mtile(c, tm), _mtile(c, tm), _mtile(c, tm)],
        out_specs=_mtile(c, tm),
        out_shape=jax.ShapeDtypeStruct((c, mm), jnp.float32),
        compiler_params=_PARALLEL,
    )(a2.reshape(c, mm), w3, b3, lf, mf, sf)

    return jnp.transpose(out.reshape(c, n, hm, wm), (1, 0, 2, 3))


# speedup vs baseline: 2.2813x; 1.0066x over previous
import functools

import jax
import jax.numpy as jnp
from jax.experimental import pallas as pl
from jax.experimental.pallas import tpu as pltpu


_PARALLEL = pltpu.CompilerParams(dimension_semantics=("parallel",))


def _ru(x, m):
    return (x + m - 1) // m * m


def _resident(shape):
    nd = len(shape)
    return pl.BlockSpec(tuple(shape), lambda t, _nd=nd: (0,) * _nd)


def _conv_body(x_ref, w_ref, sb_ref, o_ref, xs_ref, *, offsets, tm, tmh_r, relu):
    base = pl.program_id(0) * tm
    xs_ref[...] = x_ref[:, pl.ds(base, tmh_r)]
    acc = jnp.zeros(o_ref.shape, jnp.float32)
    for k, off in enumerate(offsets):
        acc += jnp.dot(w_ref[k], xs_ref[:, off:off + tm],
                       preferred_element_type=jnp.float32)
    y = acc * sb_ref[0:1, :].T + sb_ref[1:2, :].T
    if relu:
        y = jnp.maximum(y, 0.0)
    o_ref[...] = y.astype(o_ref.dtype)


def _conv_bn(x, w, scale, bias, relu=True, tm=2048):
    c, n, h, wd = x.shape
    kh, kw, cin, cout = w.shape
    pad = kh // 2
    hp, wp = h + 2 * pad, wd + 2 * pad
    mp = n * hp * wp
    l_out = _ru(mp, tm)
    nt = l_out // tm
    mr = pad * wp + pad
    tmh_r = _ru(tm + 2 * mr, 128)
    lh = mr + l_out + (tmh_r - tm - mr)

    xp = jnp.pad(x, ((0, 0), (0, 0), (pad, pad), (pad, pad))).reshape(c, mp)
    slab = jnp.pad(xp, ((0, 0), (mr, lh - mp - mr)))
    offsets = [ki * wp + kj for ki in range(kh) for kj in range(kw)]
    wf = jnp.transpose(w.reshape(kh * kw, cin, cout), (0, 2, 1)).astype(jnp.bfloat16)
    sb = jnp.stack([scale, bias]).astype(jnp.float32)

    out = pl.pallas_call(
        functools.partial(_conv_body, offsets=offsets, tm=tm, tmh_r=tmh_r,
                          relu=relu),
        grid=(nt,),
        in_specs=[_resident(slab.shape), _resident(wf.shape), _resident(sb.shape)],
        out_specs=pl.BlockSpec((cout, tm), lambda t: (0, t)),
        out_shape=jax.ShapeDtypeStruct((cout, l_out), jnp.bfloat16),
        scratch_shapes=[pltpu.VMEM((cin, tmh_r), jnp.bfloat16)],
        compiler_params=_PARALLEL,
    )(slab, wf, sb)
    out = out[:, :mp].reshape(cout, n, hp, wp)
    return out[:, :, pad:pad + h, pad:pad + wd]


def _pool_body(x_ref, se_ref, so_ref, o_ref, *, c, rows):
    w2 = x_ref.shape[2]
    xw = x_ref[...].astype(jnp.float32).reshape(c, rows, 2, w2)
    hmax = jnp.max(xw, axis=2).reshape(c * rows, w2)
    hsum = jnp.sum(xw, axis=2).reshape(c * rows, w2)
    even = jnp.dot(hmax, se_ref[...], preferred_element_type=jnp.float32)
    odd = jnp.dot(hmax, so_ref[...], preferred_element_type=jnp.float32)
    psum = jnp.dot(hsum, se_ref[...] + so_ref[...],
                   preferred_element_type=jnp.float32)
    y = jnp.maximum(even, odd) + psum * 0.25
    o_ref[...] = y.reshape(c, rows, w2 // 2).astype(o_ref.dtype)


def _pool_then_conv(x, w, scale, bias, rows=8):
    c, n, h2, w2 = x.shape
    h, wd = h2 // 2, w2 // 2
    nt = (n * h) // rows

    eye = jnp.eye(wd, dtype=jnp.float32)
    zl = jnp.zeros((wd, wd), jnp.float32)
    se = jnp.stack([eye, zl], axis=1).reshape(w2, wd)
    so = jnp.stack([zl, eye], axis=1).reshape(w2, wd)

    pooled = pl.pallas_call(
        functools.partial(_pool_body, c=c, rows=rows),
        grid=(nt,),
        in_specs=[pl.BlockSpec((c, 2 * rows, w2), lambda t: (0, t, 0)),
                  _resident(se.shape), _resident(so.shape)],
        out_specs=pl.BlockSpec((c, rows, wd), lambda t: (0, t, 0)),
        out_shape=jax.ShapeDtypeStruct((c, n * h, wd), jnp.bfloat16),
        compiler_params=_PARALLEL,
    )(x.reshape(c, n * h2, w2), se, so)
    return _conv_bn(pooled.reshape(c, n, h, wd), w, scale, bias)


def _trans0_body(l_ref, m_ref, s_ref, w_ref, sb_ref, o_ref):
    acc = jnp.dot(w_ref[0], l_ref[...], preferred_element_type=jnp.float32)
    acc += jnp.dot(w_ref[1], m_ref[...], preferred_element_type=jnp.float32)
    acc += jnp.dot(w_ref[2], s_ref[...], preferred_element_type=jnp.float32)
    y = jnp.maximum(acc * sb_ref[0:1, :].T + sb_ref[1:2, :].T, 0.0)
    o_ref[...] = y.astype(o_ref.dtype)


def _fuse_body(a_ref, w3_ref, b3_ref, l_ref, m_ref, s_ref, o_ref):
    attn = jnp.dot(w3_ref[...], a_ref[...],
                   preferred_element_type=jnp.float32) + b3_ref[...]
    amax = jnp.max(attn, axis=0, keepdims=True)
    e = jnp.exp(attn - amax)
    wgt = e / jnp.sum(e, axis=0, keepdims=True)
    o_ref[...] = (wgt[0:1, :] * l_ref[...].astype(jnp.float32)
                  + wgt[1:2, :] * m_ref[...].astype(jnp.float32)
                  + wgt[2:3, :] * s_ref[...].astype(jnp.float32))


def _mtile(c, tm):
    return pl.BlockSpec((c, tm), lambda t: (0, t))


def kernel(l, m, s, conv_l_pre_down_w, conv_l_pre_down_scale, conv_l_pre_down_bias, conv_l_post_down_w, conv_l_post_down_scale, conv_l_post_down_bias, conv_m_w, conv_m_scale, conv_m_bias, conv_s_pre_up_w, conv_s_pre_up_scale, conv_s_pre_up_bias, conv_s_post_up_w, conv_s_post_up_scale, conv_s_post_up_bias, trans0_w, trans0_scale, trans0_bias, trans1_w, trans1_scale, trans1_bias, trans2_w, trans2_scale, trans2_bias, trans3_w, trans3_scale, trans3_bias):
    n, c, hm, wm = m.shape
    lc = jnp.transpose(l, (1, 0, 2, 3)).astype(jnp.bfloat16)
    mc = jnp.transpose(m, (1, 0, 2, 3)).astype(jnp.bfloat16)
    sc = jnp.transpose(s, (1, 0, 2, 3)).astype(jnp.bfloat16)

    l1 = _conv_bn(lc, conv_l_pre_down_w, conv_l_pre_down_scale,
                  conv_l_pre_down_bias, tm=4096)
    l3 = _pool_then_conv(l1, conv_l_post_down_w, conv_l_post_down_scale,
                         conv_l_post_down_bias)
    m1 = _conv_bn(mc, conv_m_w, conv_m_scale, conv_m_bias)
    s1 = _conv_bn(sc, conv_s_pre_up_w, conv_s_pre_up_scale, conv_s_pre_up_bias)
    s2 = jnp.repeat(jnp.repeat(s1, 2, axis=2), 2, axis=3)
    s3 = _conv_bn(s2, conv_s_post_up_w, conv_s_post_up_scale, conv_s_post_up_bias)

    mm = n * hm * wm
    tm = 512
    nt = mm // tm
    lf = l3.reshape(c, mm)
    mf = m1.reshape(c, mm)
    sf = s3.reshape(c, mm)

    w0 = jnp.transpose(trans0_w.reshape(3, c, c), (0, 2, 1)).astype(jnp.bfloat16)
    sb0 = jnp.stack([trans0_scale, trans0_bias]).astype(jnp.float32)
    a0 = pl.pallas_call(
        _trans0_body,
        grid=(nt,),
        in_specs=[_mtile(c, tm), _mtile(c, tm), _mtile(c, tm),
                  _resident(w0.shape), _resident(sb0.shape)],
        out_specs=_mtile(c, tm),
        out_shape=jax.ShapeDtypeStruct((c, mm), jnp.bfloat16),
        compiler_params=_PARALLEL,
    )(lf, mf, sf, w0, sb0)

    a1 = _conv_bn(a0.reshape(c, n, hm, wm), trans1_w, trans1_scale, trans1_bias)
    a2 = _conv_bn(a1, trans2_w, trans2_scale, trans2_bias)

    w3 = (trans3_w.reshape(c, 3).T * trans3_scale.reshape(3, 1)).astype(jnp.bfloat16)
    b3 = trans3_bias.reshape(3, 1).astype(jnp.float32)
    out = pl.pallas_call(
        _fuse_body,
        grid=(nt,),
        in_specs=[_mtile(c, tm), _resident(w3.shape), _resident(b3.shape),
                  _mtile(c, tm), _mtile(c, tm), _mtile(c, tm)],
        out_specs=_mtile(c, tm),
        out_shape=jax.ShapeDtypeStruct((c, mm), jnp.float32),
        compiler_params=_PARALLEL,
    )(a2.reshape(c, mm), w3, b3, lf, mf, sf)

    return jnp.transpose(out.reshape(c, n, hm, wm), (1, 0, 2, 3))
